# baseline (speedup 1.0000x reference)
import functools

import jax
import jax.numpy as jnp
from jax import lax
from jax.experimental import pallas as pl
from jax.experimental.pallas import tpu as pltpu
from jax.experimental.pallas import tpu_sc as plsc

VOCAB = 100000
SEQ = 200
DIM = 32
BATCH = 1024

NROWS = BATCH * SEQ
NW = 32
ROWS_PER_W = NROWS // NW
CHUNK = 128
POSF = SEQ * DIM

GPS = 5
SROWS = GPS * CHUNK
NSUP = ROWS_PER_W // SROWS
NBUF = 4


def _body(x_hbm, tok_hbm, pos_hbm, out_hbm, idx_v, pos_v, buf, gsem, osem):
    wid = lax.axis_index("s") * 2 + lax.axis_index("c")
    base = wid * ROWS_PER_W

    pltpu.sync_copy(x_hbm.at[pl.ds(base, ROWS_PER_W)], idx_v)
    pltpu.sync_copy(pos_hbm, pos_v)

    def start_gathers(s, b):
        return [
            pltpu.async_copy(
                tok_hbm.at[idx_v.at[pl.ds((s * GPS + j) * CHUNK, CHUNK)]],
                buf.at[b].at[pl.ds(j * CHUNK, CHUNK)], gsem)
            for j in range(GPS)
        ]

    def add_pos(s, b):
        bufb = buf.at[b]
        p0 = (s * SROWS) % SEQ

        def run(r, _):
            t = lax.rem(p0 + r, SEQ)
            bufb[r, 0:16] = bufb[r, 0:16] + pos_v[t, 0:16]
            bufb[r, 16:32] = bufb[r, 16:32] + pos_v[t, 16:32]
            return 0

        lax.fori_loop(0, SROWS, run, 0)

    def start_store(s, b):
        return pltpu.async_copy(
            buf.at[b], out_hbm.at[pl.ds(base + s * SROWS, SROWS)], osem)

    gathers, stores = {}, {}
    for s in range(NSUP + 2):
        if 2 <= s:
            for d in gathers.pop(s - 2):
                d.wait()
            add_pos(s - 2, (s - 2) % NBUF)
            stores[s - 2] = start_store(s - 2, (s - 2) % NBUF)
        if s < NSUP:
            if s >= NBUF:
                stores.pop(s - NBUF).wait()
            gathers[s] = start_gathers(s, s % NBUF)
    for d in stores.values():
        d.wait()


@functools.partial(jax.jit, static_argnames=())
def kernel(x, token_table, pos_table):
    x_flat = x.reshape(NROWS).astype(jnp.int32)
    run = pl.kernel(
        _body,
        out_type=jax.ShapeDtypeStruct((NROWS, DIM), jnp.float32),
        mesh=plsc.VectorSubcoreMesh(core_axis_name="c", subcore_axis_name="s"),
        scratch_types=[
            pltpu.VMEM((ROWS_PER_W,), jnp.int32),
            pltpu.VMEM((SEQ, DIM), jnp.float32),
            pltpu.VMEM((NBUF, SROWS, DIM), jnp.float32),
            pltpu.SemaphoreType.DMA,
            pltpu.SemaphoreType.DMA,
        ],
        compiler_params=pltpu.CompilerParams(use_tc_tiling_on_sc=False),
    )
    out = run(x_flat, token_table, pos_table)
    return out.reshape(BATCH, SEQ, DIM)
